# Initial kernel scaffold; baseline (speedup 1.0000x reference)
import jax
import jax.numpy as jnp
from jax.experimental import pallas as pl


def kernel(x, emb, W_ih, W_hh, b_ih, b_hh, W_out, b_out):
    raise NotImplementedError("write your pallas kernel here")


# speedup vs baseline: 3.8424x; 3.8424x over previous
import jax
import jax.numpy as jnp
from jax.experimental import pallas as pl
from jax.experimental.pallas import tpu as pltpu
from jax.experimental.pallas import tpu_sc as plsc

VOCAB = 100000
EMB = 64
EMB_PAD = 128
HID = 256
NCLS = 1000
B = 4096
L = 50

GATHER_WINDOW = 128
BT = 512


def _sc_gather(emb, idx_flat):
    n = idx_flat.shape[0]
    idx2 = idx_flat.reshape(1, n)
    mesh = plsc.VectorSubcoreMesh(core_axis_name="core", subcore_axis_name="subcore")

    @pl.kernel(
        out_type=jax.ShapeDtypeStruct((n, EMB_PAD), emb.dtype),
        mesh=mesh,
    )
    def gather_kernel(emb_hbm, idx_hbm, out_hbm):
        def body(idx_vmem, out_vmem):
            pltpu.sync_copy(emb_hbm.at[idx_vmem.at[0]], out_vmem)

        pltpu.emit_pipeline(
            body,
            grid=(n // GATHER_WINDOW,),
            in_specs=[
                pl.BlockSpec((1, GATHER_WINDOW), index_map=lambda i: (0, i))
            ],
            out_specs=[
                pl.BlockSpec((GATHER_WINDOW, EMB_PAD), index_map=lambda i: (i, 0))
            ],
            core_axis_name=("core", "subcore"),
            dimension_semantics=(pltpu.PARALLEL,),
        )(idx_hbm, out_hbm)

    return gather_kernel(emb, idx2)


def _rnn_body(xe_ref, wih_ref, whh_ref, b_ref, wout_ref, bout_ref, out_ref):
    wih = wih_ref[...]
    whh = whh_ref[...]
    b = b_ref[...]

    def step(t, h):
        xt = xe_ref[t]
        return jnp.tanh(
            jnp.dot(xt, wih, preferred_element_type=jnp.float32)
            + jnp.dot(h, whh, preferred_element_type=jnp.float32)
            + b
        )

    h0 = jnp.zeros((BT, HID), dtype=jnp.float32)
    h = jax.lax.fori_loop(0, L, step, h0)
    out_ref[...] = (
        jnp.dot(h, wout_ref[...], preferred_element_type=jnp.float32)
        + bout_ref[...]
    )


def _tc_rnn(xe3, wih_t, whh_t, b2, wout_t, bout2):
    return pl.pallas_call(
        _rnn_body,
        grid=(B // BT,),
        in_specs=[
            pl.BlockSpec((L, BT, EMB_PAD), lambda i: (0, i, 0)),
            pl.BlockSpec((EMB_PAD, HID), lambda i: (0, 0)),
            pl.BlockSpec((HID, HID), lambda i: (0, 0)),
            pl.BlockSpec((1, HID), lambda i: (0, 0)),
            pl.BlockSpec((HID, NCLS), lambda i: (0, 0)),
            pl.BlockSpec((1, NCLS), lambda i: (0, 0)),
        ],
        out_specs=pl.BlockSpec((BT, NCLS), lambda i: (i, 0)),
        out_shape=jax.ShapeDtypeStruct((B, NCLS), jnp.float32),
        compiler_params=pltpu.CompilerParams(
            dimension_semantics=("parallel",),
        ),
    )(xe3, wih_t, whh_t, b2, wout_t, bout2)


def kernel(x, emb, W_ih, W_hh, b_ih, b_hh, W_out, b_out):
    idx_flat = x.T.reshape(-1).astype(jnp.int32)
    emb_pad = jnp.concatenate(
        [emb, jnp.zeros((VOCAB, EMB_PAD - EMB), emb.dtype)], axis=1
    )
    wih_pad = jnp.concatenate(
        [W_ih.T, jnp.zeros((EMB_PAD - EMB, HID), W_ih.dtype)], axis=0
    )
    xe = _sc_gather(emb_pad, idx_flat)
    xe3 = xe.reshape(L, B, EMB_PAD)
    b2 = (b_ih + b_hh).reshape(1, HID)
    bout2 = b_out.reshape(1, NCLS)
    return _tc_rnn(xe3, wih_pad, W_hh.T, b2, W_out.T, bout2)
